# baseline (speedup 1.0000x reference)
import jax
import jax.numpy as jnp
from jax import lax
from jax.experimental import pallas as pl
from jax.experimental.pallas import tpu as pltpu
from jax.experimental.pallas import tpu_sc as plsc

NUM_CLASSES = 1000
B = 1024
S = 50
C_CHUNK = 100
NCPS = NUM_CLASSES // C_CHUNK
NCHUNK = S * NCPS
BUF = C_CHUNK * B
NW = 32
MAX_ITERS = (NCHUNK + NW - 1) // NW


def _sc_body(xflat_hbm, out_hbm, xs_v, buf_v):
    wid = lax.axis_index("s") * 2 + lax.axis_index("c")
    lanes = lax.iota(jnp.int32, 16)
    ones = jnp.ones((16,), jnp.int32)
    zeros = jnp.zeros((16,), jnp.int32)

    def zero_init(k, _):
        buf_v[pl.ds(k * 16, 16)] = zeros
        return 0

    lax.fori_loop(0, BUF // 16, zero_init, 0)

    def chunk_step(t, _):
        i = wid + t * NW

        @pl.when(i < NCHUNK)
        def _():
            s = i // NCPS
            c0 = (i % NCPS) * C_CHUNK
            pltpu.sync_copy(xflat_hbm.at[pl.ds(s * B, B)], xs_v)

            def scatter(g, val16):
                vals = xs_v[pl.ds(g * 16, 16)]
                mask = (vals >= c0) & (vals < c0 + C_CHUNK)
                pos = (vals - c0) * B + g * 16 + lanes
                plsc.store_scatter(buf_v, [pos], val16, mask=mask)
                return val16

            lax.fori_loop(0, B // 16, scatter, ones)
            pltpu.sync_copy(buf_v, out_hbm.at[pl.ds(i * BUF, BUF)])
            lax.fori_loop(0, B // 16, scatter, zeros)

        return 0

    lax.fori_loop(0, MAX_ITERS, chunk_step, 0)


def kernel(x):
    xflat = x.T.reshape(S * B)
    out = pl.kernel(
        _sc_body,
        out_type=jax.ShapeDtypeStruct((S * NUM_CLASSES * B,), jnp.int32),
        mesh=plsc.VectorSubcoreMesh(core_axis_name="c", subcore_axis_name="s"),
        compiler_params=pltpu.CompilerParams(needs_layout_passes=False),
        scratch_types=[
            pltpu.VMEM((B,), jnp.int32),
            pltpu.VMEM((BUF,), jnp.int32),
        ],
    )(xflat)
    return jnp.transpose(out.reshape(S, NUM_CLASSES, B), (2, 0, 1))


# speedup vs baseline: 2.6320x; 2.6320x over previous
import jax
import jax.numpy as jnp
from jax import lax
from jax.experimental import pallas as pl
from jax.experimental.pallas import tpu as pltpu
from jax.experimental.pallas import tpu_sc as plsc

NUM_CLASSES = 1000
B = 1024
S = 50
C_CHUNK = 40
NCPS = NUM_CLASSES // C_CHUNK
NCHUNK = S * NCPS
BUF = C_CHUNK * B
NW = 32
MAX_T = (NCHUNK + NW - 1) // NW
NBUF = 2


def _sc_body(xflat_hbm, out_hbm, xs0, xs1, buf0, buf1, sem0, sem1):
    wid = lax.axis_index("s") * 2 + lax.axis_index("c")
    lanes = lax.iota(jnp.int32, 16)
    ones = jnp.ones((16,), jnp.int32)
    zeros = jnp.zeros((16,), jnp.int32)
    bufs = (buf0, buf1)
    xss = (xs0, xs1)
    sems = (sem0, sem1)

    def zero_init(k, _):
        buf0[pl.ds(k * 16, 16)] = zeros
        buf1[pl.ds(k * 16, 16)] = zeros
        return 0

    lax.fori_loop(0, BUF // 16, zero_init, 0)

    def scatter_chunk(buf, xs, c0, val16):
        def scatter(g, v):
            vals = xs[pl.ds(g * 16, 16)]
            mask = (vals >= c0) & (vals < c0 + C_CHUNK)
            c_l = vals - c0
            pos = (
                jnp.left_shift(jnp.right_shift(c_l, 3), 13)
                + jnp.left_shift(jnp.bitwise_and(c_l, 7), 7)
                + ((g >> 3) << 10) + ((g & 7) << 4)
                + lanes
            )
            plsc.store_scatter(buf, [pos], v, mask=mask)
            return v

        lax.fori_loop(0, B // 16, scatter, val16)

    def step(t, _):
        for b in range(NBUF):
            tb = t * NBUF + b
            i = wid + tb * NW
            ip = i - NBUF * NW

            @pl.when((ip >= 0) & (ip < NCHUNK))
            def _():
                pltpu.make_async_copy(
                    bufs[b], out_hbm.at[pl.ds(ip * BUF, BUF)], sems[b]
                ).wait()
                scatter_chunk(bufs[b], xss[b], (ip % NCPS) * C_CHUNK, zeros)

            @pl.when(i < NCHUNK)
            def _():
                s = i // NCPS
                c0 = (i % NCPS) * C_CHUNK
                pltpu.sync_copy(xflat_hbm.at[pl.ds(s * B, B)], xss[b])
                scatter_chunk(bufs[b], xss[b], c0, ones)
                pltpu.async_copy(
                    bufs[b], out_hbm.at[pl.ds(i * BUF, BUF)], sems[b]
                )

        return 0

    lax.fori_loop(0, MAX_T // NBUF + 2, step, 0)


def kernel(x):
    xflat = x.T.reshape(S * B)
    out = pl.kernel(
        _sc_body,
        out_type=jax.ShapeDtypeStruct((S * NUM_CLASSES * B,), jnp.int32),
        mesh=plsc.VectorSubcoreMesh(core_axis_name="c", subcore_axis_name="s"),
        compiler_params=pltpu.CompilerParams(needs_layout_passes=False),
        scratch_types=[
            pltpu.VMEM((B,), jnp.int32),
            pltpu.VMEM((B,), jnp.int32),
            pltpu.VMEM((BUF,), jnp.int32),
            pltpu.VMEM((BUF,), jnp.int32),
            pltpu.SemaphoreType.DMA,
            pltpu.SemaphoreType.DMA,
        ],
    )(xflat)
    out5 = out.reshape(S, NUM_CLASSES // 8, B // 128, 8, 128)
    return jnp.transpose(out5, (2, 4, 0, 1, 3)).reshape(B, S, NUM_CLASSES)


# speedup vs baseline: 5.3882x; 2.0472x over previous
import jax
import jax.numpy as jnp
from jax.experimental import pallas as pl
from jax.experimental.pallas import tpu as pltpu

NUM_CLASSES = 1000
B = 1024
S = 50


def _onehot_block(x_ref, o_ref):
    c = jax.lax.broadcasted_iota(jnp.int32, (1, NUM_CLASSES, B), 1)
    o_ref[...] = (c == x_ref[...]).astype(jnp.int32)


def kernel(x):
    xt = x.T.reshape(S, 1, B)
    out_t = pl.pallas_call(
        _onehot_block,
        grid=(S,),
        in_specs=[pl.BlockSpec((1, 1, B), lambda s: (s, 0, 0))],
        out_specs=pl.BlockSpec((1, NUM_CLASSES, B), lambda s: (s, 0, 0)),
        out_shape=jax.ShapeDtypeStruct((S, NUM_CLASSES, B), jnp.int32),
        compiler_params=pltpu.CompilerParams(allow_input_fusion=[True]),
    )(xt)
    return jnp.transpose(out_t, (2, 0, 1))
